# Initial kernel scaffold; baseline (speedup 1.0000x reference)
import jax
import jax.numpy as jnp
from jax.experimental import pallas as pl


def kernel(x, edge_index, W, b):
    raise NotImplementedError("write your pallas kernel here")


# speedup vs baseline: 83.8499x; 83.8499x over previous
import functools

import jax
import jax.numpy as jnp
from jax import lax
from jax.experimental import pallas as pl
from jax.experimental.pallas import tpu as pltpu
from jax.experimental.pallas import tpu_sc as plsc

N = 10000
D = 128
E = 320000

NC = 2
NS = 16
L = 16

NPAD = 12288
HALF = NPAD // NC
NT = HALF // NS
NTV = NT // L
ECHUNK = E // NS
EV = ECHUNK // L

_MESH = plsc.VectorSubcoreMesh(core_axis_name="c", subcore_axis_name="s")


def _rsqrt_f32(d):
    xi = lax.bitcast_convert_type(d, jnp.int32)
    yi = jnp.int32(0x5F3759DF) - (xi >> 1)
    y = lax.bitcast_convert_type(yi, jnp.float32)
    for _ in range(3):
        y = y * (1.5 - 0.5 * d * y * y)
    return y


def _tc_matvec(x_ref, w_ref, h_ref):
    h_ref[...] = lax.dot_general(
        x_ref[...], w_ref[...], (((1,), (0,)), ((), ())),
        preferred_element_type=jnp.float32)


@functools.partial(
    pl.kernel,
    out_type=(
        jax.ShapeDtypeStruct((NPAD,), jnp.float32),
        jax.ShapeDtypeStruct((NPAD,), jnp.float32),
    ),
    mesh=_MESH,
    scratch_types=[
        pltpu.VMEM((ECHUNK,), jnp.int32),
        pltpu.VMEM((HALF,), jnp.float32),
        pltpu.VMEM((NS * NT,), jnp.float32),
        pltpu.VMEM((NT,), jnp.float32),
        pltpu.VMEM((NT,), jnp.float32),
        pltpu.VMEM((NT,), jnp.float32),
        pltpu.VMEM_SHARED((NS * HALF,), jnp.float32),
    ],
    compiler_params=pltpu.CompilerParams(needs_layout_passes=False),
)
def _sc_degree(col_hbm, h_hbm, g_out, dis_out,
               colv, hist, red, hsl, gsl, dsl, shared):
    c = lax.axis_index("c")
    s = lax.axis_index("s")
    base = c * HALF

    zero16 = jnp.zeros((L,), jnp.float32)

    def zbody(i, carry):
        hist[pl.ds(i * L, L)] = zero16
        return carry

    lax.fori_loop(0, HALF // L, zbody, 0)

    pltpu.sync_copy(col_hbm.at[pl.ds(s * ECHUNK, ECHUNK)], colv)

    ones = jnp.ones((L,), jnp.float32)

    def body(i, carry):
        cols = colv[pl.ds(i * L, L)]
        loc = cols - base
        m = (loc >= 0) & (loc < HALF)
        idx = jnp.where(m, loc, 0)
        plsc.addupdate_scatter(hist, [idx], ones, mask=m)
        return carry

    lax.fori_loop(0, EV, body, 0)

    pltpu.sync_copy(hist, shared.at[pl.ds(s * HALF, HALF)])
    plsc.subcore_barrier()
    for t in range(NS):
        pltpu.sync_copy(shared.at[pl.ds(t * HALF + s * NT, NT)],
                        red.at[pl.ds(t * NT, NT)])
    pltpu.sync_copy(h_hbm.at[pl.ds(base + s * NT, NT)], hsl)

    def ebody(j, carry):
        acc = red[pl.ds(j * L, L)]
        for t in range(1, NS):
            acc = acc + red[pl.ds(t * NT + j * L, L)]
        d = acc + 1.0
        y = _rsqrt_f32(d)
        dsl[pl.ds(j * L, L)] = y
        gsl[pl.ds(j * L, L)] = y * hsl[pl.ds(j * L, L)]
        return carry

    lax.fori_loop(0, NTV, ebody, 0)

    pltpu.sync_copy(gsl, g_out.at[pl.ds(base + s * NT, NT)])
    pltpu.sync_copy(dsl, dis_out.at[pl.ds(base + s * NT, NT)])


@functools.partial(
    pl.kernel,
    out_type=jax.ShapeDtypeStruct((NPAD,), jnp.float32),
    mesh=_MESH,
    scratch_types=[
        pltpu.VMEM((ECHUNK,), jnp.int32),
        pltpu.VMEM((ECHUNK,), jnp.int32),
        pltpu.VMEM((NPAD,), jnp.float32),
        pltpu.VMEM((HALF,), jnp.float32),
        pltpu.VMEM((NS * NT,), jnp.float32),
        pltpu.VMEM((NT,), jnp.float32),
        pltpu.VMEM((NT,), jnp.float32),
        pltpu.VMEM((L,), jnp.float32),
        pltpu.VMEM_SHARED((NS * HALF,), jnp.float32),
    ],
    compiler_params=pltpu.CompilerParams(needs_layout_passes=False),
)
def _sc_edges(row_hbm, col_hbm, g_hbm, dis_hbm, b_hbm, out_hbm,
              rowv, colv, gv, spart, red, dsl, osl, bv, shared):
    c = lax.axis_index("c")
    s = lax.axis_index("s")
    base = c * HALF

    zero16 = jnp.zeros((L,), jnp.float32)

    def zbody(i, carry):
        spart[pl.ds(i * L, L)] = zero16
        return carry

    lax.fori_loop(0, HALF // L, zbody, 0)

    pltpu.sync_copy(g_hbm, gv)
    pltpu.sync_copy(row_hbm.at[pl.ds(s * ECHUNK, ECHUNK)], rowv)
    pltpu.sync_copy(col_hbm.at[pl.ds(s * ECHUNK, ECHUNK)], colv)

    def body(i, carry):
        rows = rowv[pl.ds(i * L, L)]
        cols = colv[pl.ds(i * L, L)]
        gvals = plsc.load_gather(gv, [rows])
        loc = cols - base
        m = (loc >= 0) & (loc < HALF)
        idx = jnp.where(m, loc, 0)
        plsc.addupdate_scatter(spart, [idx], gvals, mask=m)
        return carry

    lax.fori_loop(0, EV, body, 0)

    pltpu.sync_copy(spart, shared.at[pl.ds(s * HALF, HALF)])
    plsc.subcore_barrier()
    for t in range(NS):
        pltpu.sync_copy(shared.at[pl.ds(t * HALF + s * NT, NT)],
                        red.at[pl.ds(t * NT, NT)])
    pltpu.sync_copy(dis_hbm.at[pl.ds(base + s * NT, NT)], dsl)
    pltpu.sync_copy(b_hbm, bv)
    bval = bv[pl.ds(0, L)]

    def ebody(j, carry):
        acc = red[pl.ds(j * L, L)]
        for t in range(1, NS):
            acc = acc + red[pl.ds(t * NT + j * L, L)]
        y = dsl[pl.ds(j * L, L)]
        gg = gv[pl.ds(base + s * NT + j * L, L)]
        o = y * acc + y * gg + bval
        osl[pl.ds(j * L, L)] = jnp.maximum(o, 0.0)
        return carry

    lax.fori_loop(0, NTV, ebody, 0)

    pltpu.sync_copy(osl, out_hbm.at[pl.ds(base + s * NT, NT)])


@jax.jit
def kernel(x, edge_index, W, b):
    edge_index = edge_index.astype(jnp.int32)
    row = edge_index[0]
    col = edge_index[1]

    h = pl.pallas_call(
        _tc_matvec,
        out_shape=jax.ShapeDtypeStruct((N, 1), jnp.float32),
    )(x, W)

    hpad = jnp.pad(h[:, 0], (0, NPAD - N))
    g, dis = _sc_degree(col, hpad)

    b16 = jnp.broadcast_to(b.astype(jnp.float32).reshape(1), (L,))
    out_pad = _sc_edges(row, col, g, dis, b16)
    return out_pad[:N].reshape(N, 1)
